# baseline (speedup 1.0000x reference)
import jax
import jax.numpy as jnp
from jax import lax
from jax.experimental import pallas as pl
from jax.experimental.pallas import tpu as pltpu
from jax.experimental.pallas import tpu_sc as plsc

N_ROWS = 100000
D = 128
NSEG = 64
CHUNK = 128
N_FULL = N_ROWS // CHUNK
TAIL = N_ROWS - N_FULL * CHUNK
NC, NS = 1, 16
NW = NC * NS
MAXC = -(-N_FULL // NW)
HI = N_FULL - (MAXC - 1) * NW
NBUF = 3


def _body(nodes, ids, zeros, out,
          ibufs, bufs, tidx_v, tail_v, acc_sh, semns, semis, semscs, sem_t):
    c = lax.axis_index("c")
    s = lax.axis_index("s")
    wid = s * NC + c

    def gather(j):
        b = j % NBUF
        r0 = (wid + j * NW) * CHUNK
        pltpu.async_copy(nodes.at[pl.ds(r0, CHUNK)], bufs[b], semns[b])
        pltpu.async_copy(ids.at[pl.ds(r0, CHUNK)], ibufs[b], semis[b])

    def gather_wait(j):
        b = j % NBUF
        pltpu.make_async_copy(nodes.at[pl.ds(0, CHUNK)], bufs[b], semns[b]).wait()
        pltpu.make_async_copy(ids.at[pl.ds(0, CHUNK)], ibufs[b], semis[b]).wait()

    def scatter(j):
        b = j % NBUF
        pltpu.async_copy(bufs[b], acc_sh.at[ibufs[b]], semscs[b], add=True)

    def scatter_wait(j):
        b = j % NBUF
        pltpu.make_async_copy(bufs[b], acc_sh.at[ibufs[b]], semscs[b]).wait()

    gather(0)

    @pl.when(s == 0)
    def _init():
        pltpu.sync_copy(zeros, acc_sh)

    plsc.subcore_barrier()

    for i in range(MAXC):
        if i >= 2:
            scatter_wait(i - 2)

        def step(i=i):
            j = i + 1
            if j < MAXC:
                if j == MAXC - 1:
                    @pl.when(wid < HI)
                    def _():
                        gather(j)
                else:
                    gather(j)
            gather_wait(i)
            scatter(i)

        if i == MAXC - 1:
            @pl.when(wid < HI)
            def _():
                step()
        else:
            step()

    scatter_wait(MAXC - 2)

    @pl.when(wid < HI)
    def _last():
        scatter_wait(MAXC - 1)

    @pl.when(wid == NW - 1)
    def _tail():
        r0 = N_FULL * CHUNK
        pltpu.sync_copy(ids.at[pl.ds(r0, TAIL)], tidx_v)
        pltpu.async_copy(nodes.at[pl.ds(r0, TAIL)], tail_v, sem_t).wait()
        pltpu.sync_copy(tail_v, acc_sh.at[tidx_v], add=True)

    plsc.subcore_barrier()

    @pl.when(s == 0)
    def _flush():
        pltpu.sync_copy(acc_sh, out.at[c])


@jax.jit
def _segsum(nodes, ids32, zeros):
    mesh = plsc.VectorSubcoreMesh(core_axis_name="c", subcore_axis_name="s", num_cores=1)
    partials = pl.kernel(
        _body,
        out_type=jax.ShapeDtypeStruct((NC, NSEG, D), jnp.float32),
        mesh=mesh,
        scratch_types=[
            [pltpu.VMEM((CHUNK,), jnp.int32) for _ in range(NBUF)],
            [pltpu.VMEM((CHUNK, D), jnp.float32) for _ in range(NBUF)],
            pltpu.VMEM((TAIL,), jnp.int32),
            pltpu.VMEM((TAIL, D), jnp.float32),
            pltpu.VMEM_SHARED((NSEG, D), jnp.float32),
            [pltpu.SemaphoreType.DMA for _ in range(NBUF)],
            [pltpu.SemaphoreType.DMA for _ in range(NBUF)],
            [pltpu.SemaphoreType.DMA for _ in range(NBUF)],
            pltpu.SemaphoreType.DMA,
        ],
    )(nodes, ids32, zeros)
    return partials[0]


def kernel(nodes, segment_ids, num_graphs):
    del num_graphs
    ids32 = segment_ids.astype(jnp.int32)
    zeros = jnp.zeros((NSEG, D), jnp.float32)
    return _segsum(nodes, ids32, zeros)


# speedup vs baseline: 1.0486x; 1.0486x over previous
import jax
import jax.numpy as jnp
from jax import lax
from jax.experimental import pallas as pl
from jax.experimental.pallas import tpu as pltpu
from jax.experimental.pallas import tpu_sc as plsc

N_ROWS = 100000
D = 128
NSEG = 64
CHUNK = 128
NC, NS = 2, 16
NW = NC * NS
SPAN = 3128
LSPAN = N_ROWS - (NW - 1) * SPAN
NMAIN = SPAN // CHUNK
LMAIN = LSPAN // CHUNK
TAILA = SPAN - NMAIN * CHUNK
TAILB = LSPAN - LMAIN * CHUNK
TPADA, TPADB = 64, 96
NBUF = 3
L = 16
IBUF = SPAN + 8


def _body(nodes, ids, zeros, out,
          ibuf, idxbufs, bufs, tidxa, tidxb, taila_v, tailb_v, acc_sh,
          semns, semscs, semi, sem_t):
    c = lax.axis_index("c")
    s = lax.axis_index("s")
    wid = s * NC + c
    last = wid == NW - 1
    base = wid * SPAN

    def gather(j, b):
        pltpu.async_copy(nodes.at[pl.ds(base + j * CHUNK, CHUNK)],
                         bufs[b], semns[b])

    def gather_wait(b):
        pltpu.make_async_copy(nodes.at[pl.ds(0, CHUNK)], bufs[b], semns[b]).wait()

    def stage_idx(j, b):
        for k in range(CHUNK // L):
            idxbufs[b][pl.ds(k * L, L)] = ibuf[pl.ds(j * CHUNK + k * L, L)]

    def scatter(j, b):
        pltpu.async_copy(bufs[b], acc_sh.at[idxbufs[b]], semscs[b], add=True)

    def scatter_wait(b):
        pltpu.make_async_copy(bufs[b], acc_sh.at[idxbufs[b]], semscs[b]).wait()

    gather(0, 0)

    @pl.when(~last)
    def _ids_a():
        pltpu.async_copy(ids.at[pl.ds(base, SPAN)], ibuf.at[pl.ds(0, SPAN)], semi)

    @pl.when(last)
    def _ids_b():
        pltpu.async_copy(ids.at[pl.ds(base, LSPAN)], ibuf.at[pl.ds(0, LSPAN)], semi)

    pltpu.sync_copy(zeros.at[pl.ds(0, TPADA)], taila_v)
    pltpu.sync_copy(zeros, tailb_v)

    @pl.when(~last)
    def _taila():
        r0 = base + NMAIN * CHUNK
        pltpu.async_copy(nodes.at[pl.ds(r0, TAILA)], taila_v.at[pl.ds(0, TAILA)],
                         sem_t)

    @pl.when(last)
    def _tailb():
        r0 = base + LMAIN * CHUNK
        pltpu.async_copy(nodes.at[pl.ds(r0, TAILB)], tailb_v.at[pl.ds(0, TAILB)],
                         sem_t)

    @pl.when(s == 0)
    def _init():
        pltpu.sync_copy(zeros.at[pl.ds(0, NSEG)], acc_sh)

    plsc.subcore_barrier()

    @pl.when(~last)
    def _ids_wait_a():
        pltpu.make_async_copy(ids.at[pl.ds(0, SPAN)], ibuf.at[pl.ds(0, SPAN)],
                              semi).wait()

    @pl.when(last)
    def _ids_wait_b():
        pltpu.make_async_copy(ids.at[pl.ds(0, LSPAN)], ibuf.at[pl.ds(0, LSPAN)],
                              semi).wait()

    for i in range(NMAIN):
        if i >= 2:
            scatter_wait((i - 2) % NBUF)

        def step(i=i):
            j = i + 1
            if j < NMAIN:
                if j == NMAIN - 1:
                    @pl.when(~last)
                    def _():
                        gather(j, j % NBUF)
                else:
                    gather(j, j % NBUF)
            gather_wait(i % NBUF)
            stage_idx(i, i % NBUF)
            scatter(i, i % NBUF)

        if i == NMAIN - 1:
            @pl.when(~last)
            def _():
                step()
        else:
            step()

    scatter_wait((NMAIN - 2) % NBUF)

    @pl.when(~last)
    def _laststep():
        scatter_wait((NMAIN - 1) % NBUF)

    @pl.when(~last)
    def _tail_a():
        pltpu.make_async_copy(nodes.at[pl.ds(0, TAILA)],
                              taila_v.at[pl.ds(0, TAILA)], sem_t).wait()
        lanes = lax.iota(jnp.int32, L)
        for k in range(TPADA // L):
            v = ibuf[pl.ds(NMAIN * CHUNK + k * L, L)]
            if (k + 1) * L > TAILA:
                v = lax.select(lanes < TAILA - k * L, v, lanes * 0)
            tidxa[pl.ds(k * L, L)] = v
        pltpu.sync_copy(taila_v, acc_sh.at[tidxa], add=True)

    @pl.when(last)
    def _tail_b():
        pltpu.make_async_copy(nodes.at[pl.ds(0, TAILB)],
                              tailb_v.at[pl.ds(0, TAILB)], sem_t).wait()
        lanes = lax.iota(jnp.int32, L)
        for k in range(TPADB // L):
            v = ibuf[pl.ds(LMAIN * CHUNK + k * L, L)]
            if (k + 1) * L > TAILB:
                v = lax.select(lanes < TAILB - k * L, v, lanes * 0)
            tidxb[pl.ds(k * L, L)] = v
        pltpu.sync_copy(tailb_v, acc_sh.at[tidxb], add=True)

    plsc.subcore_barrier()

    @pl.when(s == 0)
    def _flush():
        pltpu.sync_copy(acc_sh, out.at[c])


@jax.jit
def _segsum(nodes, ids32, zeros):
    mesh = plsc.VectorSubcoreMesh(core_axis_name="c", subcore_axis_name="s")
    partials = pl.kernel(
        _body,
        out_type=jax.ShapeDtypeStruct((NC, NSEG, D), jnp.float32),
        mesh=mesh,
        scratch_types=[
            pltpu.VMEM((IBUF,), jnp.int32),
            [pltpu.VMEM((CHUNK,), jnp.int32) for _ in range(NBUF)],
            [pltpu.VMEM((CHUNK, D), jnp.float32) for _ in range(NBUF)],
            pltpu.VMEM((TPADA,), jnp.int32),
            pltpu.VMEM((TPADB,), jnp.int32),
            pltpu.VMEM((TPADA, D), jnp.float32),
            pltpu.VMEM((TPADB, D), jnp.float32),
            pltpu.VMEM_SHARED((NSEG, D), jnp.float32),
            [pltpu.SemaphoreType.DMA for _ in range(NBUF)],
            [pltpu.SemaphoreType.DMA for _ in range(NBUF)],
            pltpu.SemaphoreType.DMA,
            pltpu.SemaphoreType.DMA,
        ],
    )(nodes, ids32, zeros)
    return partials[0] + partials[1]


def kernel(nodes, segment_ids, num_graphs):
    del num_graphs
    ids32 = segment_ids.astype(jnp.int32)
    zeros = jnp.zeros((TPADB, D), jnp.float32)
    return _segsum(nodes, ids32, zeros)


# speedup vs baseline: 1.5929x; 1.5190x over previous
import jax
import jax.numpy as jnp
from jax import lax
from jax.experimental import pallas as pl
from jax.experimental.pallas import tpu as pltpu
from jax.experimental.pallas import tpu_sc as plsc

N_ROWS = 100000
D = 128
NSEG = 64
CHUNK = 128
N_FULL = N_ROWS // CHUNK
TAIL = N_ROWS - N_FULL * CHUNK
NC, NS = 2, 16
NW = NC * NS
MAXC = -(-N_FULL // NW)
HI = N_FULL - (MAXC - 1) * NW
NBUF = 4


def _body(nodes, ids, zeros, out,
          ibufs, bufs, tidx_v, tail_v, acc_sh, semns, semis, semscs, sem_t):
    c = lax.axis_index("c")
    s = lax.axis_index("s")
    wid = s * NC + c

    def gather(j):
        b = j % NBUF
        r0 = (wid + j * NW) * CHUNK
        pltpu.async_copy(nodes.at[pl.ds(r0, CHUNK)], bufs[b], semns[b])
        pltpu.async_copy(ids.at[pl.ds(r0, CHUNK)], ibufs[b], semis[b])

    def gather_wait(j):
        b = j % NBUF
        pltpu.make_async_copy(nodes.at[pl.ds(0, CHUNK)], bufs[b], semns[b]).wait()
        pltpu.make_async_copy(ids.at[pl.ds(0, CHUNK)], ibufs[b], semis[b]).wait()

    def scatter(j):
        b = j % NBUF
        pltpu.async_copy(bufs[b], acc_sh.at[ibufs[b]], semscs[b], add=True)

    def scatter_wait(j):
        b = j % NBUF
        pltpu.make_async_copy(bufs[b], acc_sh.at[ibufs[b]], semscs[b]).wait()

    gather(0)
    gather(1)

    @pl.when(s == 0)
    def _init():
        pltpu.sync_copy(zeros, acc_sh)

    plsc.subcore_barrier()

    for i in range(MAXC):
        if i >= 2:
            scatter_wait(i - 2)

        def step(i=i):
            j = i + 2
            if j < MAXC:
                if j == MAXC - 1:
                    @pl.when(wid < HI)
                    def _():
                        gather(j)
                else:
                    gather(j)
            gather_wait(i)
            scatter(i)

        if i == MAXC - 1:
            @pl.when(wid < HI)
            def _():
                step()
        else:
            step()

    scatter_wait(MAXC - 2)

    @pl.when(wid < HI)
    def _last():
        scatter_wait(MAXC - 1)

    @pl.when(wid == NW - 1)
    def _tail():
        r0 = N_FULL * CHUNK
        pltpu.sync_copy(ids.at[pl.ds(r0, TAIL)], tidx_v)
        pltpu.async_copy(nodes.at[pl.ds(r0, TAIL)], tail_v, sem_t).wait()
        pltpu.sync_copy(tail_v, acc_sh.at[tidx_v], add=True)

    plsc.subcore_barrier()

    @pl.when(s == 0)
    def _flush():
        pltpu.sync_copy(acc_sh, out.at[c])


@jax.jit
def _segsum(nodes, ids32, zeros):
    mesh = plsc.VectorSubcoreMesh(core_axis_name="c", subcore_axis_name="s")
    partials = pl.kernel(
        _body,
        out_type=jax.ShapeDtypeStruct((NC, NSEG, D), jnp.float32),
        mesh=mesh,
        scratch_types=[
            [pltpu.VMEM((CHUNK,), jnp.int32) for _ in range(NBUF)],
            [pltpu.VMEM((CHUNK, D), jnp.float32) for _ in range(NBUF)],
            pltpu.VMEM((TAIL,), jnp.int32),
            pltpu.VMEM((TAIL, D), jnp.float32),
            pltpu.VMEM_SHARED((NSEG, D), jnp.float32),
            [pltpu.SemaphoreType.DMA for _ in range(NBUF)],
            [pltpu.SemaphoreType.DMA for _ in range(NBUF)],
            [pltpu.SemaphoreType.DMA for _ in range(NBUF)],
            pltpu.SemaphoreType.DMA,
        ],
    )(nodes, ids32, zeros)
    return partials[0] + partials[1]


def kernel(nodes, segment_ids, num_graphs):
    del num_graphs
    ids32 = segment_ids.astype(jnp.int32)
    zeros = jnp.zeros((NSEG, D), jnp.float32)
    return _segsum(nodes, ids32, zeros)


# speedup vs baseline: 1.6157x; 1.0144x over previous
import jax
import jax.numpy as jnp
from jax import lax
from jax.experimental import pallas as pl
from jax.experimental.pallas import tpu as pltpu
from jax.experimental.pallas import tpu_sc as plsc

N_ROWS = 100000
D = 128
NSEG = 64
CHUNK = 128
N_FULL = N_ROWS // CHUNK
TAIL = N_ROWS - N_FULL * CHUNK
NC, NS = 2, 16
NW = NC * NS
MAXC = -(-N_FULL // NW)
HI = N_FULL - (MAXC - 1) * NW
NBUF = 6


def _body(nodes, ids, zeros, out,
          ibufs, bufs, tidx_v, tail_v, acc_sh, semns, semis, semscs, sem_t):
    c = lax.axis_index("c")
    s = lax.axis_index("s")
    wid = s * NC + c

    def gather(j):
        b = j % NBUF
        r0 = (wid + j * NW) * CHUNK
        pltpu.async_copy(nodes.at[pl.ds(r0, CHUNK)], bufs[b], semns[b])
        pltpu.async_copy(ids.at[pl.ds(r0, CHUNK)], ibufs[b], semis[b])

    def gather_wait(j):
        b = j % NBUF
        pltpu.make_async_copy(nodes.at[pl.ds(0, CHUNK)], bufs[b], semns[b]).wait()
        pltpu.make_async_copy(ids.at[pl.ds(0, CHUNK)], ibufs[b], semis[b]).wait()

    def scatter(j):
        b = j % NBUF
        pltpu.async_copy(bufs[b], acc_sh.at[ibufs[b]], semscs[b], add=True)

    def scatter_wait(j):
        b = j % NBUF
        pltpu.make_async_copy(bufs[b], acc_sh.at[ibufs[b]], semscs[b]).wait()

    for j0 in range(NBUF - 2):
        gather(j0)

    @pl.when(s == 0)
    def _init():
        pltpu.sync_copy(zeros, acc_sh)

    plsc.subcore_barrier()

    for i in range(MAXC):
        if i >= 2:
            scatter_wait(i - 2)

        def step(i=i):
            j = i + NBUF - 2
            if j < MAXC:
                if j == MAXC - 1:
                    @pl.when(wid < HI)
                    def _():
                        gather(j)
                else:
                    gather(j)
            gather_wait(i)
            scatter(i)

        if i == MAXC - 1:
            @pl.when(wid < HI)
            def _():
                step()
        else:
            step()

    scatter_wait(MAXC - 2)

    @pl.when(wid < HI)
    def _last():
        scatter_wait(MAXC - 1)

    @pl.when(wid == NW - 1)
    def _tail():
        r0 = N_FULL * CHUNK
        pltpu.sync_copy(ids.at[pl.ds(r0, TAIL)], tidx_v)
        pltpu.async_copy(nodes.at[pl.ds(r0, TAIL)], tail_v, sem_t).wait()
        pltpu.sync_copy(tail_v, acc_sh.at[tidx_v], add=True)

    plsc.subcore_barrier()

    @pl.when(s == 0)
    def _flush():
        pltpu.sync_copy(acc_sh, out.at[c])


@jax.jit
def _segsum(nodes, ids32, zeros):
    mesh = plsc.VectorSubcoreMesh(core_axis_name="c", subcore_axis_name="s")
    partials = pl.kernel(
        _body,
        out_type=jax.ShapeDtypeStruct((NC, NSEG, D), jnp.float32),
        mesh=mesh,
        scratch_types=[
            [pltpu.VMEM((CHUNK,), jnp.int32) for _ in range(NBUF)],
            [pltpu.VMEM((CHUNK, D), jnp.float32) for _ in range(NBUF)],
            pltpu.VMEM((TAIL,), jnp.int32),
            pltpu.VMEM((TAIL, D), jnp.float32),
            pltpu.VMEM_SHARED((NSEG, D), jnp.float32),
            [pltpu.SemaphoreType.DMA for _ in range(NBUF)],
            [pltpu.SemaphoreType.DMA for _ in range(NBUF)],
            [pltpu.SemaphoreType.DMA for _ in range(NBUF)],
            pltpu.SemaphoreType.DMA,
        ],
    )(nodes, ids32, zeros)
    return partials[0] + partials[1]


def kernel(nodes, segment_ids, num_graphs):
    del num_graphs
    ids32 = segment_ids.astype(jnp.int32)
    zeros = jnp.zeros((NSEG, D), jnp.float32)
    return _segsum(nodes, ids32, zeros)
